# baseline (device time: 559092 ns/iter reference)
import jax
import jax.numpy as jnp
from jax import lax
from jax.experimental import pallas as pl
from jax.experimental.pallas import tpu as pltpu

S = 2048
K = 4096
N = 8192
HALF = S // 2
NT = 16
NTS = N // NT


def kernel(O, Wo):
    A = O.reshape(S, K).astype(jnp.bfloat16)
    x = lax.axis_index("x")
    xs = jnp.reshape(x.astype(jnp.int32), (1,))

    def body(s_ref, a_ref, w_ref, out_ref, recv_ref,
             send_buf, recv_vmem, send_sems, recv_sems, local_sem):
        n = pl.program_id(0)
        h = pl.program_id(1)
        my_x = s_ref[0]
        peer = (1 - my_x, lax.axis_index("y"), lax.axis_index("z"))

        @pl.when(jnp.logical_and(h == 0, n == 0))
        def _():
            barrier = pltpu.get_barrier_semaphore()
            pl.semaphore_signal(barrier, inc=1, device_id=peer,
                                device_id_type=pl.DeviceIdType.MESH)
            pl.semaphore_wait(barrier, 1)

        acc = jnp.dot(a_ref[...], w_ref[...].astype(jnp.bfloat16),
                      preferred_element_type=jnp.float32)

        def send_desc(slot, nn):
            return pltpu.make_async_remote_copy(
                src_ref=send_buf.at[slot],
                dst_ref=recv_ref.at[:, pl.ds(nn * NTS, NTS)],
                send_sem=send_sems.at[slot],
                recv_sem=recv_sems.at[nn],
                device_id=peer,
                device_id_type=pl.DeviceIdType.MESH,
            )

        @pl.when(h == 0)
        def _():
            for nn in range(NT):
                @pl.when(n == nn)
                def _(nn=nn):
                    slot = nn % 2
                    if nn >= 2:
                        send_desc(slot, nn - 2).wait_send()
                    send_buf[slot, :, :] = acc
                    send_desc(slot, nn).start()

        @pl.when(h == 1)
        def _():
            for nn in range(NT):
                @pl.when(n == nn)
                def _(nn=nn):
                    if nn == NT - 1:
                        send_desc(0, NT - 2).wait_send()
                        send_desc(1, NT - 1).wait_send()
                    send_desc(0, nn).wait_recv()
                    cp = pltpu.make_async_copy(
                        recv_ref.at[:, pl.ds(nn * NTS, NTS)],
                        recv_vmem, local_sem)
                    cp.start()
                    cp.wait()
                    out_ref[...] = acc + recv_vmem[...]

    grid_spec = pltpu.PrefetchScalarGridSpec(
        num_scalar_prefetch=1,
        grid=(NT, 2),
        in_specs=[
            pl.BlockSpec(
                (HALF, K),
                lambda n, h, s: (jnp.where(h == 0, 1 - s[0], s[0]), 0)),
            pl.BlockSpec((K, NTS), lambda n, h, s: (0, n)),
        ],
        out_specs=[
            pl.BlockSpec((HALF, NTS), lambda n, h, s: (0, n)),
            pl.BlockSpec(memory_space=pl.ANY),
        ],
        scratch_shapes=[
            pltpu.VMEM((2, HALF, NTS), jnp.float32),
            pltpu.VMEM((HALF, NTS), jnp.float32),
            pltpu.SemaphoreType.DMA((2,)),
            pltpu.SemaphoreType.DMA((NT,)),
            pltpu.SemaphoreType.DMA,
        ],
    )
    out, _ = pl.pallas_call(
        body,
        grid_spec=grid_spec,
        out_shape=[
            jax.ShapeDtypeStruct((HALF, N), jnp.float32),
            jax.ShapeDtypeStruct((HALF, N), jnp.float32),
        ],
        compiler_params=pltpu.CompilerParams(
            collective_id=0,
            dimension_semantics=("arbitrary", "arbitrary"),
        ),
    )(xs, A, Wo)
    return out.reshape(1, HALF, N)


# device time: 433999 ns/iter; 1.2882x vs baseline; 1.2882x over previous
import jax
import jax.numpy as jnp
from jax import lax
from jax.experimental import pallas as pl
from jax.experimental.pallas import tpu as pltpu

S = 2048
K = 4096
N = 8192
HALF = S // 2
NT = 16
NTS = N // NT
LAG = 2


def kernel(O, Wo):
    A = O.reshape(S, K).astype(jnp.bfloat16)
    x = lax.axis_index("x")
    xs = jnp.reshape(x.astype(jnp.int32), (1,))

    def body(s_ref, a_any, w_ref, out_ref, recv_ref,
             a_vmem, send_buf, stash, recv_vmem,
             send_sems, recv_sems, local_sem):
        n = pl.program_id(0)
        my_x = s_ref[0]
        peer = (1 - my_x, lax.axis_index("y"), lax.axis_index("z"))
        my_start = my_x * HALF
        peer_start = (1 - my_x) * HALF

        @pl.when(n == 0)
        def _():
            barrier = pltpu.get_barrier_semaphore()
            pl.semaphore_signal(barrier, inc=1, device_id=peer,
                                device_id_type=pl.DeviceIdType.MESH)
            pl.semaphore_wait(barrier, 1)
            cp = pltpu.make_async_copy(a_any, a_vmem, local_sem)
            cp.start()
            cp.wait()

        wt = w_ref[...].astype(jnp.bfloat16)
        peer_acc = jnp.dot(a_vmem[pl.ds(peer_start, HALF), :], wt,
                           preferred_element_type=jnp.float32)
        my_acc = jnp.dot(a_vmem[pl.ds(my_start, HALF), :], wt,
                         preferred_element_type=jnp.float32)

        def send_desc(slot, nn):
            return pltpu.make_async_remote_copy(
                src_ref=send_buf.at[slot],
                dst_ref=recv_ref.at[:, pl.ds(nn * NTS, NTS)],
                send_sem=send_sems.at[slot],
                recv_sem=recv_sems.at[nn],
                device_id=peer,
                device_id_type=pl.DeviceIdType.MESH,
            )

        for nn in range(NT + LAG):
            @pl.when(n == nn)
            def _(nn=nn):
                if nn < NT:
                    slot = nn % 2
                    if nn >= 2:
                        send_desc(slot, nn - 2).wait_send()
                    send_buf[slot, :, :] = peer_acc
                    stash[nn % (LAG + 1), :, :] = my_acc
                    send_desc(slot, nn).start()
                else:
                    send_desc((nn - LAG) % 2, nn - LAG).wait_send()
                if nn >= LAG:
                    m = nn - LAG
                    send_desc(0, m).wait_recv()
                    cp = pltpu.make_async_copy(
                        recv_ref.at[:, pl.ds(m * NTS, NTS)],
                        recv_vmem, local_sem)
                    cp.start()
                    cp.wait()
                    out_ref[...] = stash[m % (LAG + 1)] + recv_vmem[...]

    grid_spec = pltpu.PrefetchScalarGridSpec(
        num_scalar_prefetch=1,
        grid=(NT + LAG,),
        in_specs=[
            pl.BlockSpec(memory_space=pl.ANY),
            pl.BlockSpec((K, NTS),
                         lambda n, s: (0, jnp.minimum(n, NT - 1))),
        ],
        out_specs=[
            pl.BlockSpec((HALF, NTS),
                         lambda n, s: (0, jnp.clip(n - LAG, 0, NT - 1))),
            pl.BlockSpec(memory_space=pl.ANY),
        ],
        scratch_shapes=[
            pltpu.VMEM((S, K), jnp.bfloat16),
            pltpu.VMEM((2, HALF, NTS), jnp.float32),
            pltpu.VMEM((LAG + 1, HALF, NTS), jnp.float32),
            pltpu.VMEM((HALF, NTS), jnp.float32),
            pltpu.SemaphoreType.DMA((2,)),
            pltpu.SemaphoreType.DMA((NT,)),
            pltpu.SemaphoreType.DMA,
        ],
    )
    out, _ = pl.pallas_call(
        body,
        grid_spec=grid_spec,
        out_shape=[
            jax.ShapeDtypeStruct((HALF, N), jnp.float32),
            jax.ShapeDtypeStruct((HALF, N), jnp.float32),
        ],
        compiler_params=pltpu.CompilerParams(
            collective_id=0,
            dimension_semantics=("arbitrary",),
            vmem_limit_bytes=60 * 1024 * 1024,
        ),
    )(xs, A, Wo)
    return out.reshape(1, HALF, N)


# device time: 253528 ns/iter; 2.2052x vs baseline; 1.7118x over previous
import jax
import jax.numpy as jnp
from jax import lax
from jax.experimental import pallas as pl
from jax.experimental.pallas import tpu as pltpu

S = 2048
K = 4096
N = 8192
HALF = S // 2
NT = 16
NTS = N // NT
LAG = 2


def kernel(O, Wo):
    A = O.reshape(S, K).astype(jnp.bfloat16)
    x = lax.axis_index("x")
    xs = jnp.reshape(x.astype(jnp.int32), (1,))

    def body(s_ref, a_any, w_ref, out_ref,
             a_my, a_peer, send_buf, stash, recv_slots,
             send_sems, recv_sems, local_sem):
        n = pl.program_id(0)
        my_x = s_ref[0]
        peer = (1 - my_x, lax.axis_index("y"), lax.axis_index("z"))
        my_start = my_x * HALF
        peer_start = (1 - my_x) * HALF

        @pl.when(n == 0)
        def _():
            barrier = pltpu.get_barrier_semaphore()
            pl.semaphore_signal(barrier, inc=1, device_id=peer,
                                device_id_type=pl.DeviceIdType.MESH)
            pl.semaphore_wait(barrier, 1)
            cp = pltpu.make_async_copy(
                a_any.at[pl.ds(my_start, HALF), :], a_my, local_sem)
            cp.start()
            cp.wait()
            cp = pltpu.make_async_copy(
                a_any.at[pl.ds(peer_start, HALF), :], a_peer, local_sem)
            cp.start()
            cp.wait()

        wt = w_ref[...].astype(jnp.bfloat16)
        peer_acc = jnp.dot(a_peer[...], wt,
                           preferred_element_type=jnp.float32)
        my_acc = jnp.dot(a_my[...], wt,
                         preferred_element_type=jnp.float32)

        def send_desc(slot, nn):
            return pltpu.make_async_remote_copy(
                src_ref=send_buf.at[slot],
                dst_ref=recv_slots.at[nn],
                send_sem=send_sems.at[slot],
                recv_sem=recv_sems.at[nn],
                device_id=peer,
                device_id_type=pl.DeviceIdType.MESH,
            )

        for nn in range(NT + LAG):
            @pl.when(n == nn)
            def _(nn=nn):
                if nn < NT:
                    slot = nn % 2
                    if nn >= 2:
                        send_desc(slot, nn - 2).wait_send()
                    send_buf[slot, :, :] = peer_acc.astype(jnp.bfloat16)
                    stash[nn % (LAG + 1), :, :] = my_acc.astype(jnp.bfloat16)
                    send_desc(slot, nn).start()
                else:
                    send_desc((nn - LAG) % 2, nn - LAG).wait_send()
                if nn >= LAG:
                    m = nn - LAG
                    send_desc(0, m).wait_recv()
                    out_ref[...] = (
                        stash[m % (LAG + 1)].astype(jnp.float32)
                        + recv_slots[m].astype(jnp.float32))

    grid_spec = pltpu.PrefetchScalarGridSpec(
        num_scalar_prefetch=1,
        grid=(NT + LAG,),
        in_specs=[
            pl.BlockSpec(memory_space=pl.ANY),
            pl.BlockSpec((K, NTS),
                         lambda n, s: (0, jnp.minimum(n, NT - 1))),
        ],
        out_specs=pl.BlockSpec(
            (HALF, NTS), lambda n, s: (0, jnp.clip(n - LAG, 0, NT - 1))),
        scratch_shapes=[
            pltpu.VMEM((HALF, K), jnp.bfloat16),
            pltpu.VMEM((HALF, K), jnp.bfloat16),
            pltpu.VMEM((2, HALF, NTS), jnp.bfloat16),
            pltpu.VMEM((LAG + 1, HALF, NTS), jnp.bfloat16),
            pltpu.VMEM((NT, HALF, NTS), jnp.bfloat16),
            pltpu.SemaphoreType.DMA((2,)),
            pltpu.SemaphoreType.DMA((NT,)),
            pltpu.SemaphoreType.DMA,
        ],
    )
    out = pl.pallas_call(
        body,
        grid_spec=grid_spec,
        out_shape=jax.ShapeDtypeStruct((HALF, N), jnp.float32),
        compiler_params=pltpu.CompilerParams(
            collective_id=0,
            dimension_semantics=("arbitrary",),
            vmem_limit_bytes=64 * 1024 * 1024,
        ),
    )(xs, A, Wo)
    return out.reshape(1, HALF, N)
